# baseline (speedup 1.0000x reference)
import functools

import jax
import jax.numpy as jnp
from jax import lax
from jax.experimental import pallas as pl
from jax.experimental.pallas import tpu as pltpu
from jax.experimental.pallas import tpu_sc as plsc

N_NODES = 10000
DIM = 128
NUM_CORES = 2
NUM_SUBCORES = 16
NUM_WORKERS = NUM_CORES * NUM_SUBCORES
CHUNK = 128
ACC_ROWS = 10240
ZERO_ROWS_PER_TILE = ACC_ROWS // NUM_SUBCORES
OUT_ROW_STRIDE = 624
OUT_ROWS_PER_TILE = 640


def _sc_agg(h, src3, dst3):
    chunks_per_worker = src3.shape[1]
    group = next(g for g in range(min(40, chunks_per_worker), 0, -1)
                 if chunks_per_worker % g == 0)
    num_groups = chunks_per_worker // group
    mesh = plsc.VectorSubcoreMesh(core_axis_name="c", subcore_axis_name="s")

    @functools.partial(
        pl.kernel,
        out_type=jax.ShapeDtypeStruct((NUM_CORES * N_NODES, DIM), jnp.float32),
        mesh=mesh,
        scratch_types=[
            pltpu.VMEM((group, CHUNK), jnp.int32),
            pltpu.VMEM((group, CHUNK), jnp.int32),
            pltpu.VMEM((2, CHUNK, DIM), jnp.float32),
            pltpu.VMEM_SHARED((ACC_ROWS, DIM), jnp.float32),
            pltpu.SemaphoreType.DMA,
        ],
    )
    def agg_kernel(h_hbm, src_hbm, dst_hbm, out_hbm, src_v, dst_v, rows_v, acc_sh, sem):
        cid = lax.axis_index("c")
        sid = lax.axis_index("s")
        wid = cid * NUM_SUBCORES + sid

        def zero_body(i, carry):
            r = i // 8
            c = lax.rem(i, 8) * 16
            rows_v[0, r, pl.ds(c, 16)] = jnp.zeros((16,), jnp.float32)
            return carry
        lax.fori_loop(0, CHUNK * (DIM // 16), zero_body, 0)
        base = sid * ZERO_ROWS_PER_TILE
        for off in range(0, ZERO_ROWS_PER_TILE, CHUNK):
            pltpu.sync_copy(rows_v.at[0], acc_sh.at[pl.ds(base + off, CHUNK)])
        plsc.subcore_barrier()

        def group_body(g, carry):
            pltpu.sync_copy(src_hbm.at[wid].at[pl.ds(g * group, group)], src_v)
            pltpu.sync_copy(dst_hbm.at[wid].at[pl.ds(g * group, group)], dst_v)
            def edge_body(j, carry2):
                p = lax.rem(j, 2)
                pltpu.async_copy(h_hbm.at[src_v.at[j]], rows_v.at[p], sem).wait()
                pltpu.sync_copy(rows_v.at[p], acc_sh.at[dst_v.at[j]], add=True)
                return carry2
            lax.fori_loop(0, group, edge_body, 0)
            return carry
        lax.fori_loop(0, num_groups, group_body, 0)
        plsc.subcore_barrier()

        row0 = sid * OUT_ROW_STRIDE
        pltpu.sync_copy(
            acc_sh.at[pl.ds(row0, OUT_ROWS_PER_TILE)],
            out_hbm.at[pl.ds(cid * N_NODES + row0, OUT_ROWS_PER_TILE)],
        )

    return agg_kernel(h, src3, dst3)


def _row_block_specs(rows):
    return pl.BlockSpec((rows, DIM), lambda i: (i, 0))


def _full_spec(shape):
    return pl.BlockSpec(shape, lambda i: (0,) * len(shape))


def _mlp_layer(h, a0, a1, w1, b1, w2, b2):
    rows = 2000

    def body(h_ref, a0_ref, a1_ref, w1_ref, b1_ref, w2_ref, b2_ref, o_ref):
        z = h_ref[...] + a0_ref[...] + a1_ref[...]
        t = jnp.dot(z, w1_ref[...], preferred_element_type=jnp.float32) + b1_ref[...]
        t = jnp.maximum(t, 0.0)
        o = jnp.dot(t, w2_ref[...], preferred_element_type=jnp.float32) + b2_ref[...]
        o_ref[...] = jnp.maximum(o, 0.0)

    return pl.pallas_call(
        body,
        grid=(N_NODES // rows,),
        in_specs=[
            _row_block_specs(rows), _row_block_specs(rows), _row_block_specs(rows),
            _full_spec((DIM, DIM)), _full_spec((1, DIM)),
            _full_spec((DIM, DIM)), _full_spec((1, DIM)),
        ],
        out_specs=_row_block_specs(rows),
        out_shape=jax.ShapeDtypeStruct((N_NODES, DIM), jnp.float32),
    )(h, a0, a1, w1, b1.reshape(1, DIM), w2, b2.reshape(1, DIM))


def _mlp_head(h, a0, a1, w1, b1, w2, b2, w_out, b_out):
    rows = 2000

    def body(h_ref, a0_ref, a1_ref, w1_ref, b1_ref, w2_ref, b2_ref,
             wo_ref, bo_ref, o_ref):
        z = h_ref[...] + a0_ref[...] + a1_ref[...]
        t = jnp.dot(z, w1_ref[...], preferred_element_type=jnp.float32) + b1_ref[...]
        t = jnp.maximum(t, 0.0)
        o = jnp.dot(t, w2_ref[...], preferred_element_type=jnp.float32) + b2_ref[...]
        o = jnp.maximum(o, 0.0)
        o_ref[...] = jnp.dot(o, wo_ref[...], preferred_element_type=jnp.float32) + bo_ref[...]

    return pl.pallas_call(
        body,
        grid=(N_NODES // rows,),
        in_specs=[
            _row_block_specs(rows), _row_block_specs(rows), _row_block_specs(rows),
            _full_spec((DIM, DIM)), _full_spec((1, DIM)),
            _full_spec((DIM, DIM)), _full_spec((1, DIM)),
            _full_spec((DIM, 1)), _full_spec((1, 1)),
        ],
        out_specs=pl.BlockSpec((rows, 1), lambda i: (i, 0)),
        out_shape=jax.ShapeDtypeStruct((N_NODES, 1), jnp.float32),
    )(h, a0, a1, w1, b1.reshape(1, DIM), w2, b2.reshape(1, DIM),
      w_out, b_out.reshape(1, 1))


@jax.jit
def kernel(x, edge_index,
           w1_0, b1_0, w2_0, b2_0,
           w1_1, b1_1, w2_1, b2_1,
           w1_2, b1_2, w2_2, b2_2,
           w_out, b_out):
    src = edge_index[0]
    dst = edge_index[1]
    num_edges = src.shape[0]

    epw = NUM_WORKERS * CHUNK
    e_pad = -(-num_edges // epw) * epw
    pad = e_pad - num_edges
    pad_ids = lax.rem(jnp.arange(pad, dtype=jnp.int32), jnp.int32(N_NODES))
    src_p = jnp.concatenate([src, pad_ids])
    dst_p = jnp.concatenate(
        [dst, N_NODES + lax.rem(jnp.arange(pad, dtype=jnp.int32), jnp.int32(16))])
    src3 = src_p.reshape(NUM_WORKERS, -1, CHUNK)
    dst3 = dst_p.reshape(NUM_WORKERS, -1, CHUNK)

    layers = [(w1_0, b1_0, w2_0, b2_0),
              (w1_1, b1_1, w2_1, b2_1)]
    h = x
    for (w1, b1, w2, b2) in layers:
        parts = _sc_agg(h, src3, dst3)
        a = parts.reshape(NUM_CORES, N_NODES, DIM)
        h = _mlp_layer(h, a[0], a[1], w1, b1, w2, b2)

    parts = _sc_agg(h, src3, dst3)
    a = parts.reshape(NUM_CORES, N_NODES, DIM)
    head = _mlp_head(h, a[0], a[1], w1_2, b1_2, w2_2, b2_2, w_out, b_out)
    return head.squeeze(-1)


# speedup vs baseline: 1.4740x; 1.4740x over previous
import functools

import jax
import jax.numpy as jnp
from jax import lax
from jax.experimental import pallas as pl
from jax.experimental.pallas import tpu as pltpu
from jax.experimental.pallas import tpu_sc as plsc

N_NODES = 10000
DIM = 128
NUM_CORES = 2
NUM_SUBCORES = 16
NUM_WORKERS = NUM_CORES * NUM_SUBCORES
CHUNK = 128
GROUP = 2
STAGE = 2048
ACC_ROWS = 10240
ZERO_ROWS_PER_TILE = ACC_ROWS // NUM_SUBCORES
OUT_ROW_STRIDE = 624
OUT_ROWS_PER_TILE = 640


def _sc_agg(h, src3, dst3):
    edges_per_worker = src3.shape[1]
    blk = GROUP * CHUNK
    assert edges_per_worker % STAGE == 0 and STAGE % blk == 0
    num_stages = edges_per_worker // STAGE
    mesh = plsc.VectorSubcoreMesh(core_axis_name="c", subcore_axis_name="s")

    @functools.partial(
        pl.kernel,
        out_type=jax.ShapeDtypeStruct((NUM_CORES * N_NODES, DIM), jnp.float32),
        mesh=mesh,
        scratch_types=[
            pltpu.VMEM((STAGE,), jnp.int32),
            pltpu.VMEM((STAGE,), jnp.int32),
            pltpu.VMEM((GROUP * CHUNK, DIM), jnp.float32),
            pltpu.VMEM_SHARED((ACC_ROWS, DIM), jnp.float32),
            pltpu.SemaphoreType.DMA,
        ],
    )
    def agg_kernel(h_hbm, src_hbm, dst_hbm, out_hbm, src_v, dst_v, rows_v, acc_sh,
                   sem):
        cid = lax.axis_index("c")
        sid = lax.axis_index("s")
        wid = cid * NUM_SUBCORES + sid

        def zero_body(i, carry):
            r = i // 8
            c = lax.rem(i, 8) * 16
            rows_v[r, pl.ds(c, 16)] = jnp.zeros((16,), jnp.float32)
            return carry
        lax.fori_loop(0, CHUNK * (DIM // 16), zero_body, 0)
        base = sid * ZERO_ROWS_PER_TILE
        for off in range(0, ZERO_ROWS_PER_TILE, CHUNK):
            pltpu.sync_copy(rows_v.at[pl.ds(0, CHUNK)],
                            acc_sh.at[pl.ds(base + off, CHUNK)])
        plsc.subcore_barrier()

        def stage_body(s, carry):
            pltpu.sync_copy(src_hbm.at[wid].at[pl.ds(s * STAGE, STAGE)], src_v)
            pltpu.sync_copy(dst_hbm.at[wid].at[pl.ds(s * STAGE, STAGE)], dst_v)

            def group_body(g, c2):
                idx = pl.ds(g * blk, blk)
                pltpu.async_copy(h_hbm.at[src_v.at[idx]], rows_v, sem).wait()
                pltpu.sync_copy(rows_v, acc_sh.at[dst_v.at[idx]], add=True)
                return c2
            lax.fori_loop(0, STAGE // blk, group_body, 0)
            return carry
        lax.fori_loop(0, num_stages, stage_body, 0)
        plsc.subcore_barrier()

        row0 = sid * OUT_ROW_STRIDE
        pltpu.sync_copy(
            acc_sh.at[pl.ds(row0, OUT_ROWS_PER_TILE)],
            out_hbm.at[pl.ds(cid * N_NODES + row0, OUT_ROWS_PER_TILE)],
        )

    return agg_kernel(h, src3, dst3)


def _row_block_specs(rows):
    return pl.BlockSpec((rows, DIM), lambda i: (i, 0))


def _full_spec(shape):
    return pl.BlockSpec(shape, lambda i: (0,) * len(shape))


def _mlp_layer(h, a0, a1, w1, b1, w2, b2):
    rows = 2000

    def body(h_ref, a0_ref, a1_ref, w1_ref, b1_ref, w2_ref, b2_ref, o_ref):
        z = h_ref[...] + a0_ref[...] + a1_ref[...]
        t = jnp.dot(z, w1_ref[...], preferred_element_type=jnp.float32) + b1_ref[...]
        t = jnp.maximum(t, 0.0)
        o = jnp.dot(t, w2_ref[...], preferred_element_type=jnp.float32) + b2_ref[...]
        o_ref[...] = jnp.maximum(o, 0.0)

    return pl.pallas_call(
        body,
        grid=(N_NODES // rows,),
        in_specs=[
            _row_block_specs(rows), _row_block_specs(rows), _row_block_specs(rows),
            _full_spec((DIM, DIM)), _full_spec((1, DIM)),
            _full_spec((DIM, DIM)), _full_spec((1, DIM)),
        ],
        out_specs=_row_block_specs(rows),
        out_shape=jax.ShapeDtypeStruct((N_NODES, DIM), jnp.float32),
    )(h, a0, a1, w1, b1.reshape(1, DIM), w2, b2.reshape(1, DIM))


def _mlp_head(h, a0, a1, w1, b1, w2, b2, w_out, b_out):
    rows = 2000

    def body(h_ref, a0_ref, a1_ref, w1_ref, b1_ref, w2_ref, b2_ref,
             wo_ref, bo_ref, o_ref):
        z = h_ref[...] + a0_ref[...] + a1_ref[...]
        t = jnp.dot(z, w1_ref[...], preferred_element_type=jnp.float32) + b1_ref[...]
        t = jnp.maximum(t, 0.0)
        o = jnp.dot(t, w2_ref[...], preferred_element_type=jnp.float32) + b2_ref[...]
        o = jnp.maximum(o, 0.0)
        o_ref[...] = jnp.dot(o, wo_ref[...], preferred_element_type=jnp.float32) + bo_ref[...]

    return pl.pallas_call(
        body,
        grid=(N_NODES // rows,),
        in_specs=[
            _row_block_specs(rows), _row_block_specs(rows), _row_block_specs(rows),
            _full_spec((DIM, DIM)), _full_spec((1, DIM)),
            _full_spec((DIM, DIM)), _full_spec((1, DIM)),
            _full_spec((DIM, 1)), _full_spec((1, 1)),
        ],
        out_specs=pl.BlockSpec((rows, 1), lambda i: (i, 0)),
        out_shape=jax.ShapeDtypeStruct((N_NODES, 1), jnp.float32),
    )(h, a0, a1, w1, b1.reshape(1, DIM), w2, b2.reshape(1, DIM),
      w_out, b_out.reshape(1, 1))


@jax.jit
def kernel(x, edge_index,
           w1_0, b1_0, w2_0, b2_0,
           w1_1, b1_1, w2_1, b2_1,
           w1_2, b1_2, w2_2, b2_2,
           w_out, b_out):
    src = edge_index[0]
    dst = edge_index[1]
    num_edges = src.shape[0]

    epw = NUM_WORKERS * STAGE
    e_pad = -(-num_edges // epw) * epw
    pad = e_pad - num_edges
    pad_ids = lax.rem(jnp.arange(pad, dtype=jnp.int32), jnp.int32(N_NODES))
    src_p = jnp.concatenate([src, pad_ids])
    dump_rows = ACC_ROWS - N_NODES
    dst_p = jnp.concatenate(
        [dst,
         N_NODES + lax.rem(jnp.arange(pad, dtype=jnp.int32),
                           jnp.int32(dump_rows))])
    src3 = src_p.reshape(NUM_WORKERS, -1)
    dst3 = dst_p.reshape(NUM_WORKERS, -1)

    layers = [(w1_0, b1_0, w2_0, b2_0),
              (w1_1, b1_1, w2_1, b2_1)]
    h = x
    for (w1, b1, w2, b2) in layers:
        parts = _sc_agg(h, src3, dst3)
        a = parts.reshape(NUM_CORES, N_NODES, DIM)
        h = _mlp_layer(h, a[0], a[1], w1, b1, w2, b2)

    parts = _sc_agg(h, src3, dst3)
    a = parts.reshape(NUM_CORES, N_NODES, DIM)
    head = _mlp_head(h, a[0], a[1], w1_2, b1_2, w2_2, b2_2, w_out, b_out)
    return head.squeeze(-1)


# speedup vs baseline: 1.6318x; 1.1071x over previous
import functools

import jax
import jax.numpy as jnp
from jax import lax
from jax.experimental import pallas as pl
from jax.experimental.pallas import tpu as pltpu
from jax.experimental.pallas import tpu_sc as plsc

N_NODES = 10000
DIM = 128
NUM_CORES = 2
NUM_SUBCORES = 16
NUM_WORKERS = NUM_CORES * NUM_SUBCORES
CHUNK = 128
GROUP = 2
STAGE = 2048
ACC_ROWS = 10240
ZERO_ROWS_PER_TILE = ACC_ROWS // NUM_SUBCORES
OUT_ROW_STRIDE = 624
OUT_ROWS_PER_TILE = 640


def _sc_agg(h, src3, dst3):
    edges_per_worker = src3.shape[1]
    blk = GROUP * CHUNK
    assert edges_per_worker % STAGE == 0 and STAGE % blk == 0
    num_stages = edges_per_worker // STAGE
    mesh = plsc.VectorSubcoreMesh(core_axis_name="c", subcore_axis_name="s")

    @functools.partial(
        pl.kernel,
        out_type=jax.ShapeDtypeStruct((NUM_CORES * N_NODES, DIM), jnp.float32),
        mesh=mesh,
        scratch_types=[
            pltpu.VMEM((STAGE,), jnp.int32),
            pltpu.VMEM((STAGE,), jnp.int32),
            pltpu.VMEM((2, CHUNK, DIM), jnp.float32),
            pltpu.VMEM_SHARED((ACC_ROWS, DIM), jnp.float32),
            pltpu.SemaphoreType.DMA,
            pltpu.SemaphoreType.DMA,
            pltpu.SemaphoreType.DMA,
            pltpu.SemaphoreType.DMA,
        ],
    )
    def agg_kernel(h_hbm, src_hbm, dst_hbm, out_hbm, src_v, dst_v, rows_v, acc_sh,
                   gsem0, gsem1, ssem0, ssem1):
        cid = lax.axis_index("c")
        sid = lax.axis_index("s")
        wid = cid * NUM_SUBCORES + sid

        def zero_body(i, carry):
            r = i // 8
            c = lax.rem(i, 8) * 16
            rows_v[0, r, pl.ds(c, 16)] = jnp.zeros((16,), jnp.float32)
            return carry
        lax.fori_loop(0, CHUNK * (DIM // 16), zero_body, 0)
        base = sid * ZERO_ROWS_PER_TILE
        for off in range(0, ZERO_ROWS_PER_TILE, CHUNK):
            pltpu.sync_copy(rows_v.at[0], acc_sh.at[pl.ds(base + off, CHUNK)])
        plsc.subcore_barrier()

        gsems = (gsem0, gsem1)
        ssems = (ssem0, ssem1)
        n_chunks = STAGE // CHUNK

        def stage_body(s, carry):
            pltpu.sync_copy(src_hbm.at[wid].at[pl.ds(s * STAGE, STAGE)], src_v)
            pltpu.sync_copy(dst_hbm.at[wid].at[pl.ds(s * STAGE, STAGE)], dst_v)

            gathers = [None] * n_chunks
            scatters = [None] * n_chunks

            def issue_gather(k):
                b = k % 2
                gathers[k] = pltpu.async_copy(
                    h_hbm.at[src_v.at[pl.ds(k * CHUNK, CHUNK)]],
                    rows_v.at[b], gsems[b])

            issue_gather(0)
            for k in range(n_chunks):
                b = k % 2
                gathers[k].wait()
                if k + 1 < n_chunks:
                    if k >= 1:
                        scatters[k - 1].wait()
                    issue_gather(k + 1)
                scatters[k] = pltpu.async_copy(
                    rows_v.at[b], acc_sh.at[dst_v.at[pl.ds(k * CHUNK, CHUNK)]],
                    ssems[b], add=True)
            scatters[n_chunks - 2].wait()
            scatters[n_chunks - 1].wait()
            return carry
        lax.fori_loop(0, num_stages, stage_body, 0)
        plsc.subcore_barrier()

        row0 = sid * OUT_ROW_STRIDE
        pltpu.sync_copy(
            acc_sh.at[pl.ds(row0, OUT_ROWS_PER_TILE)],
            out_hbm.at[pl.ds(cid * N_NODES + row0, OUT_ROWS_PER_TILE)],
        )

    return agg_kernel(h, src3, dst3)


def _row_block_specs(rows):
    return pl.BlockSpec((rows, DIM), lambda i: (i, 0))


def _full_spec(shape):
    return pl.BlockSpec(shape, lambda i: (0,) * len(shape))


def _mlp_layer(h, a0, a1, w1, b1, w2, b2):
    rows = 2000

    def body(h_ref, a0_ref, a1_ref, w1_ref, b1_ref, w2_ref, b2_ref, o_ref):
        z = h_ref[...] + a0_ref[...] + a1_ref[...]
        t = jnp.dot(z, w1_ref[...], preferred_element_type=jnp.float32) + b1_ref[...]
        t = jnp.maximum(t, 0.0)
        o = jnp.dot(t, w2_ref[...], preferred_element_type=jnp.float32) + b2_ref[...]
        o_ref[...] = jnp.maximum(o, 0.0)

    return pl.pallas_call(
        body,
        grid=(N_NODES // rows,),
        in_specs=[
            _row_block_specs(rows), _row_block_specs(rows), _row_block_specs(rows),
            _full_spec((DIM, DIM)), _full_spec((1, DIM)),
            _full_spec((DIM, DIM)), _full_spec((1, DIM)),
        ],
        out_specs=_row_block_specs(rows),
        out_shape=jax.ShapeDtypeStruct((N_NODES, DIM), jnp.float32),
    )(h, a0, a1, w1, b1.reshape(1, DIM), w2, b2.reshape(1, DIM))


def _mlp_head(h, a0, a1, w1, b1, w2, b2, w_out, b_out):
    rows = 2000

    def body(h_ref, a0_ref, a1_ref, w1_ref, b1_ref, w2_ref, b2_ref,
             wo_ref, bo_ref, o_ref):
        z = h_ref[...] + a0_ref[...] + a1_ref[...]
        t = jnp.dot(z, w1_ref[...], preferred_element_type=jnp.float32) + b1_ref[...]
        t = jnp.maximum(t, 0.0)
        o = jnp.dot(t, w2_ref[...], preferred_element_type=jnp.float32) + b2_ref[...]
        o = jnp.maximum(o, 0.0)
        o_ref[...] = jnp.dot(o, wo_ref[...], preferred_element_type=jnp.float32) + bo_ref[...]

    return pl.pallas_call(
        body,
        grid=(N_NODES // rows,),
        in_specs=[
            _row_block_specs(rows), _row_block_specs(rows), _row_block_specs(rows),
            _full_spec((DIM, DIM)), _full_spec((1, DIM)),
            _full_spec((DIM, DIM)), _full_spec((1, DIM)),
            _full_spec((DIM, 1)), _full_spec((1, 1)),
        ],
        out_specs=pl.BlockSpec((rows, 1), lambda i: (i, 0)),
        out_shape=jax.ShapeDtypeStruct((N_NODES, 1), jnp.float32),
    )(h, a0, a1, w1, b1.reshape(1, DIM), w2, b2.reshape(1, DIM),
      w_out, b_out.reshape(1, 1))


@jax.jit
def kernel(x, edge_index,
           w1_0, b1_0, w2_0, b2_0,
           w1_1, b1_1, w2_1, b2_1,
           w1_2, b1_2, w2_2, b2_2,
           w_out, b_out):
    src = edge_index[0]
    dst = edge_index[1]
    num_edges = src.shape[0]

    epw = NUM_WORKERS * STAGE
    e_pad = -(-num_edges // epw) * epw
    pad = e_pad - num_edges
    pad_ids = lax.rem(jnp.arange(pad, dtype=jnp.int32), jnp.int32(N_NODES))
    src_p = jnp.concatenate([src, pad_ids])
    dump_rows = ACC_ROWS - N_NODES
    dst_p = jnp.concatenate(
        [dst,
         N_NODES + lax.rem(jnp.arange(pad, dtype=jnp.int32),
                           jnp.int32(dump_rows))])
    src3 = src_p.reshape(NUM_WORKERS, -1)
    dst3 = dst_p.reshape(NUM_WORKERS, -1)

    layers = [(w1_0, b1_0, w2_0, b2_0),
              (w1_1, b1_1, w2_1, b2_1)]
    h = x
    for (w1, b1, w2, b2) in layers:
        parts = _sc_agg(h, src3, dst3)
        a = parts.reshape(NUM_CORES, N_NODES, DIM)
        h = _mlp_layer(h, a[0], a[1], w1, b1, w2, b2)

    parts = _sc_agg(h, src3, dst3)
    a = parts.reshape(NUM_CORES, N_NODES, DIM)
    head = _mlp_head(h, a[0], a[1], w1_2, b1_2, w2_2, b2_2, w_out, b_out)
    return head.squeeze(-1)


# speedup vs baseline: 1.6966x; 1.0397x over previous
import functools

import jax
import jax.numpy as jnp
from jax import lax
from jax.experimental import pallas as pl
from jax.experimental.pallas import tpu as pltpu
from jax.experimental.pallas import tpu_sc as plsc

N_NODES = 10000
DIM = 128
NUM_CORES = 2
NUM_SUBCORES = 16
NUM_WORKERS = NUM_CORES * NUM_SUBCORES
CHUNK = 128
GROUP = 2
STAGE = 2048
ACC_ROWS = 10240
ZERO_ROWS_PER_TILE = ACC_ROWS // NUM_SUBCORES
OUT_ROW_STRIDE = 624
OUT_ROWS_PER_TILE = 640


def _sc_agg(h, src3, dst3):
    edges_per_worker = src3.shape[1]
    assert edges_per_worker % STAGE == 0 and STAGE % CHUNK == 0
    num_stages = edges_per_worker // STAGE
    n_chunks = STAGE // CHUNK
    total_chunks = num_stages * n_chunks
    mesh = plsc.VectorSubcoreMesh(core_axis_name="c", subcore_axis_name="s")

    @functools.partial(
        pl.kernel,
        out_type=jax.ShapeDtypeStruct((NUM_CORES * N_NODES, DIM), jnp.float32),
        mesh=mesh,
        scratch_types=[
            pltpu.VMEM((2, STAGE), jnp.int32),
            pltpu.VMEM((2, STAGE), jnp.int32),
            pltpu.VMEM((2, CHUNK, DIM), jnp.float32),
            pltpu.VMEM_SHARED((ACC_ROWS, DIM), jnp.float32),
            pltpu.SemaphoreType.DMA,
            pltpu.SemaphoreType.DMA,
            pltpu.SemaphoreType.DMA,
            pltpu.SemaphoreType.DMA,
            pltpu.SemaphoreType.DMA,
            pltpu.SemaphoreType.DMA,
        ],
    )
    def agg_kernel(h_hbm, src_hbm, dst_hbm, out_hbm, src_v, dst_v, rows_v, acc_sh,
                   gsem0, gsem1, ssem0, ssem1, isem0, isem1):
        cid = lax.axis_index("c")
        sid = lax.axis_index("s")
        wid = cid * NUM_SUBCORES + sid

        gsems = (gsem0, gsem1)
        ssems = (ssem0, ssem1)

        idx_h = [None] * num_stages

        def issue_idx(s):
            b = s % 2
            idx_h[s] = (
                pltpu.async_copy(src_hbm.at[wid].at[pl.ds(s * STAGE, STAGE)],
                                 src_v.at[b], isem0),
                pltpu.async_copy(dst_hbm.at[wid].at[pl.ds(s * STAGE, STAGE)],
                                 dst_v.at[b], isem1),
            )

        issue_idx(0)

        def zero_body(i, carry):
            r = i // 8
            c = lax.rem(i, 8) * 16
            rows_v[0, r, pl.ds(c, 16)] = jnp.zeros((16,), jnp.float32)
            return carry
        lax.fori_loop(0, CHUNK * (DIM // 16), zero_body, 0)
        base = sid * ZERO_ROWS_PER_TILE
        zh = []
        for j, off in enumerate(range(0, ZERO_ROWS_PER_TILE, CHUNK)):
            zh.append(pltpu.async_copy(
                rows_v.at[0], acc_sh.at[pl.ds(base + off, CHUNK)], ssems[j % 2]))
        for hcopy in zh:
            hcopy.wait()
        plsc.subcore_barrier()

        gathers = [None] * total_chunks
        scatters = [None] * total_chunks

        def issue_gather(t):
            s, k = divmod(t, n_chunks)
            gathers[t] = pltpu.async_copy(
                h_hbm.at[src_v.at[s % 2].at[pl.ds(k * CHUNK, CHUNK)]],
                rows_v.at[t % 2], gsems[t % 2])

        idx_h[0][0].wait()
        idx_h[0][1].wait()
        issue_gather(0)
        for t in range(total_chunks):
            b = t % 2
            s, k = divmod(t, n_chunks)
            gathers[t].wait()
            if t + 1 < total_chunks:
                if t >= 1:
                    scatters[t - 1].wait()
                if k == 0 and s + 1 < num_stages:
                    issue_idx(s + 1)
                if (t + 1) % n_chunks == 0:
                    s_next = (t + 1) // n_chunks
                    idx_h[s_next][0].wait()
                    idx_h[s_next][1].wait()
                issue_gather(t + 1)
            scatters[t] = pltpu.async_copy(
                rows_v.at[b],
                acc_sh.at[dst_v.at[s % 2].at[pl.ds(k * CHUNK, CHUNK)]],
                ssems[b], add=True)
        scatters[total_chunks - 2].wait()
        scatters[total_chunks - 1].wait()
        plsc.subcore_barrier()

        row0 = sid * OUT_ROW_STRIDE
        pltpu.sync_copy(
            acc_sh.at[pl.ds(row0, OUT_ROWS_PER_TILE)],
            out_hbm.at[pl.ds(cid * N_NODES + row0, OUT_ROWS_PER_TILE)],
        )

    return agg_kernel(h, src3, dst3)


def _row_block_specs(rows):
    return pl.BlockSpec((rows, DIM), lambda i: (i, 0))


def _full_spec(shape):
    return pl.BlockSpec(shape, lambda i: (0,) * len(shape))


def _mlp_layer(h, a0, a1, w1, b1, w2, b2):
    rows = 2000

    def body(h_ref, a0_ref, a1_ref, w1_ref, b1_ref, w2_ref, b2_ref, o_ref):
        z = h_ref[...] + a0_ref[...] + a1_ref[...]
        t = jnp.dot(z, w1_ref[...], preferred_element_type=jnp.float32) + b1_ref[...]
        t = jnp.maximum(t, 0.0)
        o = jnp.dot(t, w2_ref[...], preferred_element_type=jnp.float32) + b2_ref[...]
        o_ref[...] = jnp.maximum(o, 0.0)

    return pl.pallas_call(
        body,
        grid=(N_NODES // rows,),
        in_specs=[
            _row_block_specs(rows), _row_block_specs(rows), _row_block_specs(rows),
            _full_spec((DIM, DIM)), _full_spec((1, DIM)),
            _full_spec((DIM, DIM)), _full_spec((1, DIM)),
        ],
        out_specs=_row_block_specs(rows),
        out_shape=jax.ShapeDtypeStruct((N_NODES, DIM), jnp.float32),
    )(h, a0, a1, w1, b1.reshape(1, DIM), w2, b2.reshape(1, DIM))


def _mlp_head(h, a0, a1, w1, b1, w2, b2, w_out, b_out):
    rows = 2000

    def body(h_ref, a0_ref, a1_ref, w1_ref, b1_ref, w2_ref, b2_ref,
             wo_ref, bo_ref, o_ref):
        z = h_ref[...] + a0_ref[...] + a1_ref[...]
        t = jnp.dot(z, w1_ref[...], preferred_element_type=jnp.float32) + b1_ref[...]
        t = jnp.maximum(t, 0.0)
        o = jnp.dot(t, w2_ref[...], preferred_element_type=jnp.float32) + b2_ref[...]
        o = jnp.maximum(o, 0.0)
        o_ref[...] = jnp.dot(o, wo_ref[...], preferred_element_type=jnp.float32) + bo_ref[...]

    return pl.pallas_call(
        body,
        grid=(N_NODES // rows,),
        in_specs=[
            _row_block_specs(rows), _row_block_specs(rows), _row_block_specs(rows),
            _full_spec((DIM, DIM)), _full_spec((1, DIM)),
            _full_spec((DIM, DIM)), _full_spec((1, DIM)),
            _full_spec((DIM, 1)), _full_spec((1, 1)),
        ],
        out_specs=pl.BlockSpec((rows, 1), lambda i: (i, 0)),
        out_shape=jax.ShapeDtypeStruct((N_NODES, 1), jnp.float32),
    )(h, a0, a1, w1, b1.reshape(1, DIM), w2, b2.reshape(1, DIM),
      w_out, b_out.reshape(1, 1))


@jax.jit
def kernel(x, edge_index,
           w1_0, b1_0, w2_0, b2_0,
           w1_1, b1_1, w2_1, b2_1,
           w1_2, b1_2, w2_2, b2_2,
           w_out, b_out):
    src = edge_index[0]
    dst = edge_index[1]
    num_edges = src.shape[0]

    epw = NUM_WORKERS * STAGE
    e_pad = -(-num_edges // epw) * epw
    pad = e_pad - num_edges
    pad_ids = lax.rem(jnp.arange(pad, dtype=jnp.int32), jnp.int32(N_NODES))
    src_p = jnp.concatenate([src, pad_ids])
    dump_rows = ACC_ROWS - N_NODES
    dst_p = jnp.concatenate(
        [dst,
         N_NODES + lax.rem(jnp.arange(pad, dtype=jnp.int32),
                           jnp.int32(dump_rows))])
    src3 = src_p.reshape(NUM_WORKERS, -1)
    dst3 = dst_p.reshape(NUM_WORKERS, -1)

    layers = [(w1_0, b1_0, w2_0, b2_0),
              (w1_1, b1_1, w2_1, b2_1)]
    h = x
    for (w1, b1, w2, b2) in layers:
        parts = _sc_agg(h, src3, dst3)
        a = parts.reshape(NUM_CORES, N_NODES, DIM)
        h = _mlp_layer(h, a[0], a[1], w1, b1, w2, b2)

    parts = _sc_agg(h, src3, dst3)
    a = parts.reshape(NUM_CORES, N_NODES, DIM)
    head = _mlp_head(h, a[0], a[1], w1_2, b1_2, w2_2, b2_2, w_out, b_out)
    return head.squeeze(-1)
